# baseline (speedup 1.0000x reference)
import functools

import jax
import jax.numpy as jnp
from jax import lax
from jax.experimental import pallas as pl
from jax.experimental.pallas import tpu as pltpu
from jax.experimental.pallas import tpu_sc as plsc

_N_NODES = 10000
_N_EDGES = 40000
_IN_F = 64
_OUT_F = 64
_N_REL = 16

_INFO = plsc.get_sparse_core_info()
_NC, _NS = _INFO.num_cores, _INFO.num_subcores
_NW = _NC * _NS
_E_PAD = 40960
_B_PER_W = _E_PAD // _NW
_CHUNK = 128
_N_CHUNKS = _B_PER_W // _CHUNK


def _mm_body(nodes_ref, k_ref, out_ref):
    out_ref[...] = jnp.dot(
        nodes_ref[...], k_ref[0], preferred_element_type=jnp.float32
    ).astype(jnp.bfloat16)


def _build_table(nodes, kernels):
    return pl.pallas_call(
        _mm_body,
        grid=(_N_REL,),
        in_specs=[
            pl.BlockSpec((_N_NODES, _IN_F), lambda t: (0, 0)),
            pl.BlockSpec((1, _IN_F, _OUT_F), lambda t: (t, 0, 0)),
        ],
        out_specs=pl.BlockSpec((_N_NODES, _OUT_F), lambda t: (t, 0)),
        out_shape=jax.ShapeDtypeStruct((_N_REL * _N_NODES, _OUT_F), jnp.bfloat16),
    )(nodes, kernels)


def _sc_body(table_hbm, senders_hbm, types_hbm, out_hbm,
             s_v, t_v, idx_v, rows_v, sem):
    wid = lax.axis_index("s") * _NC + lax.axis_index("c")
    base = wid * _B_PER_W
    pltpu.sync_copy(senders_hbm.at[pl.ds(base, _B_PER_W)], s_v)
    pltpu.sync_copy(types_hbm.at[pl.ds(base, _B_PER_W)], t_v)

    def idx_body(i, _):
        sl = pl.ds(i * 16, 16)
        idx_v[sl] = t_v[sl] * _N_NODES + s_v[sl]
        return 0

    lax.fori_loop(0, _B_PER_W // 16, idx_body, 0)

    copies = [
        pltpu.async_copy(
            table_hbm.at[idx_v.at[pl.ds(j * _CHUNK, _CHUNK)]],
            rows_v.at[pl.ds(j * _CHUNK, _CHUNK)],
            sem,
        )
        for j in range(_N_CHUNKS)
    ]
    for c in copies:
        c.wait()
    pltpu.sync_copy(rows_v, out_hbm.at[pl.ds(base, _B_PER_W)])


_sc_gather = functools.partial(
    pl.kernel,
    out_type=jax.ShapeDtypeStruct((_E_PAD, _OUT_F), jnp.bfloat16),
    mesh=plsc.VectorSubcoreMesh(core_axis_name="c", subcore_axis_name="s"),
    scratch_types=[
        pltpu.VMEM((_B_PER_W,), jnp.int32),
        pltpu.VMEM((_B_PER_W,), jnp.int32),
        pltpu.VMEM((_B_PER_W,), jnp.int32),
        pltpu.VMEM((_B_PER_W, _OUT_F), jnp.bfloat16),
        pltpu.SemaphoreType.DMA,
    ],
    compiler_params=pltpu.CompilerParams(use_tc_tiling_on_sc=False),
)(_sc_body)


def kernel(nodes, senders, edge_types, kernels):
    table = _build_table(nodes, kernels)

    pad = _E_PAD - _N_EDGES
    senders_p = jnp.concatenate([senders, jnp.zeros((pad,), jnp.int32)])
    types_p = jnp.concatenate([edge_types, jnp.zeros((pad,), jnp.int32)])

    out = _sc_gather(table, senders_p, types_p)
    return out[:_N_EDGES].astype(jnp.float32)


# speedup vs baseline: 1.8916x; 1.8916x over previous
import functools

import jax
import jax.numpy as jnp
from jax import lax
from jax.experimental import pallas as pl
from jax.experimental.pallas import tpu as pltpu
from jax.experimental.pallas import tpu_sc as plsc

_N_NODES = 10000
_N_EDGES = 40000
_IN_F = 64
_OUT_F = 64
_N_REL = 16

_INFO = plsc.get_sparse_core_info()
_NC, _NS = _INFO.num_cores, _INFO.num_subcores
_NW = _NC * _NS
_B_PER_W = 1280
_TAIL = _N_EDGES - (_NW - 1) * _B_PER_W
_CHUNK = 128


def _mm_body(nodes_ref, k2_ref, out_ref):
    out_ref[...] = jnp.dot(
        nodes_ref[...], k2_ref[...], preferred_element_type=jnp.float32
    )


def _build_table(nodes, k2):
    rows_blk = 2000
    return pl.pallas_call(
        _mm_body,
        grid=(_N_NODES // rows_blk,),
        in_specs=[
            pl.BlockSpec((rows_blk, _IN_F), lambda i: (i, 0)),
            pl.BlockSpec((_IN_F, _N_REL * _OUT_F), lambda i: (0, 0)),
        ],
        out_specs=pl.BlockSpec((rows_blk, _N_REL * _OUT_F), lambda i: (i, 0)),
        out_shape=jax.ShapeDtypeStruct((_N_NODES, _N_REL * _OUT_F), jnp.float32),
    )(nodes, k2)


def _do_range(table_hbm, senders_hbm, types_hbm, out_hbm,
              s_v, t_v, idx_v, rows_v, sem, base, n_rows):
    pltpu.sync_copy(senders_hbm.at[pl.ds(base, n_rows)], s_v.at[pl.ds(0, n_rows)])
    pltpu.sync_copy(types_hbm.at[pl.ds(base, n_rows)], t_v.at[pl.ds(0, n_rows)])

    def idx_body(i, _):
        sl = pl.ds(i * 16, 16)
        idx_v[sl] = s_v[sl] * _N_REL + t_v[sl]
        return 0

    lax.fori_loop(0, n_rows // 16, idx_body, 0)

    copies = []
    off = 0
    while off < n_rows:
        c = min(_CHUNK, n_rows - off)
        copies.append(pltpu.async_copy(
            table_hbm.at[idx_v.at[pl.ds(off, c)]],
            rows_v.at[pl.ds(off, c)],
            sem,
        ))
        off += c
    for c in copies:
        c.wait()
    pltpu.sync_copy(rows_v.at[pl.ds(0, n_rows)], out_hbm.at[pl.ds(base, n_rows)])


def _sc_body(table_hbm, senders_hbm, types_hbm, out_hbm,
             s_v, t_v, idx_v, rows_v, sem):
    wid = lax.axis_index("s") * _NC + lax.axis_index("c")
    base = wid * _B_PER_W

    @pl.when(wid < _NW - 1)
    def _full():
        _do_range(table_hbm, senders_hbm, types_hbm, out_hbm,
                  s_v, t_v, idx_v, rows_v, sem, base, _B_PER_W)

    @pl.when(wid == _NW - 1)
    def _tail():
        _do_range(table_hbm, senders_hbm, types_hbm, out_hbm,
                  s_v, t_v, idx_v, rows_v, sem, base, _TAIL)


_sc_gather = functools.partial(
    pl.kernel,
    out_type=jax.ShapeDtypeStruct((_N_EDGES, _OUT_F), jnp.float32),
    mesh=plsc.VectorSubcoreMesh(core_axis_name="c", subcore_axis_name="s"),
    scratch_types=[
        pltpu.VMEM((_B_PER_W,), jnp.int32),
        pltpu.VMEM((_B_PER_W,), jnp.int32),
        pltpu.VMEM((_B_PER_W,), jnp.int32),
        pltpu.VMEM((_B_PER_W, _OUT_F), jnp.float32),
        pltpu.SemaphoreType.DMA,
    ],
    compiler_params=pltpu.CompilerParams(use_tc_tiling_on_sc=False),
)(_sc_body)


def kernel(nodes, senders, edge_types, kernels):
    k2 = kernels.transpose(1, 0, 2).reshape(_IN_F, _N_REL * _OUT_F)
    table = _build_table(nodes, k2).reshape(_N_NODES * _N_REL, _OUT_F)
    return _sc_gather(table, senders, edge_types)
